# baseline (speedup 1.0000x reference)
import functools

import jax
import jax.numpy as jnp
from jax import lax
from jax.experimental import pallas as pl
from jax.experimental.pallas import tpu as pltpu
from jax.experimental.pallas import tpu_sc as plsc

B = 256
C = 768
HW = 196
NUM_CLASSES = 1000
E = 16
TOP_K = 2

_INFO = plsc.get_sparse_core_info()
NC = _INFO.num_cores
NS = _INFO.num_subcores
L = _INFO.num_lanes
NW = NC * NS
BPW = B // NW
CCH = 128
NCH = C // CCH
NG = CCH // L


def _sc_pool_body(x_hbm, out_hbm, buf, outbuf):
    wid = lax.axis_index("s") * NC + lax.axis_index("c")
    iota = lax.iota(jnp.int32, L)
    inv_hw = jnp.float32(1.0 / HW)

    def per_b(i, carry):
        b = wid * BPW + i

        def per_chunk(ch, carry2):
            pltpu.sync_copy(x_hbm.at[b, pl.ds(ch * CCH * HW, CCH * HW)], buf)

            def per_group(g, carry3):
                base = (g * L + iota) * HW
                accs = []
                for k in range(4):
                    acc = jnp.zeros((L,), jnp.float32)
                    for s in range(k, HW, 4):
                        acc = acc + plsc.load_gather(buf, [base + s])
                    accs.append(acc)
                total = (accs[0] + accs[1]) + (accs[2] + accs[3])
                outbuf[pl.ds(ch * CCH + g * L, L)] = total * inv_hw
                return carry3

            lax.fori_loop(0, NG, per_group, 0)
            return carry2

        lax.fori_loop(0, NCH, per_chunk, 0)
        pltpu.sync_copy(outbuf, out_hbm.at[b])
        return carry

    lax.fori_loop(0, BPW, per_b, 0)


@functools.partial(
    pl.kernel,
    mesh=plsc.VectorSubcoreMesh(core_axis_name="c", subcore_axis_name="s"),
    out_type=jax.ShapeDtypeStruct((B, C), jnp.float32),
    scratch_types=[
        pltpu.VMEM((CCH * HW,), jnp.float32),
        pltpu.VMEM((C,), jnp.float32),
    ],
    compiler_params=pltpu.CompilerParams(needs_layout_passes=False),
)
def _sc_pool(x_hbm, out_hbm, buf, outbuf):
    _sc_pool_body(x_hbm, out_hbm, buf, outbuf)


def _finish_body(pooled_ref, task_ref, wt_ref, brow_ref,
                 logits_ref, weights_ref, sel_ref, mask_ref):
    t = task_ref[...]
    t = t - jnp.max(t, axis=-1, keepdims=True)
    te = jnp.exp(t)
    tsm = te / jnp.sum(te, axis=-1, keepdims=True)

    fused = jnp.concatenate([pooled_ref[...], tsm], axis=1)
    logits = jnp.dot(fused, wt_ref[...],
                     preferred_element_type=jnp.float32) + brow_ref[...]
    logits_ref[...] = logits

    m = jnp.max(logits, axis=-1, keepdims=True)
    pe = jnp.exp(logits - m)
    probs = pe / jnp.sum(pe, axis=-1, keepdims=True)
    lane = lax.broadcasted_iota(jnp.int32, (B, E), 1)
    v1 = jnp.max(probs, axis=-1, keepdims=True)
    i1 = jnp.min(jnp.where(probs == v1, lane, E), axis=-1, keepdims=True)
    p2 = jnp.where(lane == i1, -jnp.inf, probs)
    v2 = jnp.max(p2, axis=-1, keepdims=True)
    i2 = jnp.min(jnp.where(p2 == v2, lane, E), axis=-1, keepdims=True)
    s12 = v1 + v2
    weights_ref[...] = jnp.concatenate([v1 / s12, v2 / s12], axis=1)
    sel_ref[...] = jnp.concatenate([i1, i2], axis=1)

    selT = jnp.transpose(jnp.concatenate([i1, i2], axis=1), (1, 0))
    e3 = lax.broadcasted_iota(jnp.int32, (E, TOP_K, B), 0)
    mask_ref[...] = (e3 == selT.reshape(1, TOP_K, B)).astype(jnp.int32)


@jax.jit
def _run(hidden_states, task_cls, wt, brow):
    x2 = hidden_states.reshape(B, C * HW)
    pooled = _sc_pool(x2)
    return pl.pallas_call(
        _finish_body,
        out_shape=[
            jax.ShapeDtypeStruct((B, E), jnp.float32),
            jax.ShapeDtypeStruct((B, TOP_K), jnp.float32),
            jax.ShapeDtypeStruct((B, TOP_K), jnp.int32),
            jax.ShapeDtypeStruct((E, TOP_K, B), jnp.int32),
        ],
    )(pooled, task_cls, wt, brow)


def kernel(hidden_states, task_cls, W, b):
    logits, weights, sel, mask = _run(hidden_states, task_cls,
                                      W.T, b.reshape(1, E))
    return (logits, weights, sel, mask)


# speedup vs baseline: 1.1127x; 1.1127x over previous
import functools

import jax
import jax.numpy as jnp
from jax import lax
from jax.experimental import pallas as pl
from jax.experimental.pallas import tpu as pltpu
from jax.experimental.pallas import tpu_sc as plsc

B = 256
C = 768
HW = 196
NUM_CLASSES = 1000
E = 16
TOP_K = 2

_INFO = plsc.get_sparse_core_info()
NC = _INFO.num_cores
NS = _INFO.num_subcores
L = _INFO.num_lanes
NW = NC * NS
BPW = B // NW
CCH = 128
NCH = C // CCH
NG = CCH // L


NT = BPW * NCH


def _sc_pool_body(x_hbm, out_hbm, buf0, buf1, outbuf, sem0, sem1):
    wid = lax.axis_index("s") * NC + lax.axis_index("c")
    iota = lax.iota(jnp.int32, L)
    inv_hw = jnp.float32(1.0 / HW)

    def src(t):
        b = wid * BPW + t // NCH
        return x_hbm.at[b, pl.ds((t % NCH) * CCH * HW, CCH * HW)]

    def compute(t, buf):
        def per_group(g, carry):
            base = (g * L + iota) * HW
            accs = []
            for k in range(4):
                acc = jnp.zeros((L,), jnp.float32)
                for s in range(k, HW, 4):
                    acc = acc + plsc.load_gather(buf, [base + s])
                accs.append(acc)
            total = (accs[0] + accs[1]) + (accs[2] + accs[3])
            outbuf[pl.ds((t % NCH) * CCH + g * L, L)] = total * inv_hw
            return carry

        lax.fori_loop(0, NG, per_group, 0)

        @pl.when(t % NCH == NCH - 1)
        def _flush():
            pltpu.sync_copy(outbuf, out_hbm.at[wid * BPW + t // NCH])

    pltpu.async_copy(src(0), buf0, sem0)

    def ring(u, carry):
        t0 = 2 * u
        pltpu.make_async_copy(src(t0), buf0, sem0).wait()
        pltpu.async_copy(src(t0 + 1), buf1, sem1)
        compute(t0, buf0)

        @pl.when(u < NT // 2 - 1)
        def _prefetch():
            pltpu.async_copy(src(t0 + 2), buf0, sem0)

        pltpu.make_async_copy(src(t0 + 1), buf1, sem1).wait()
        compute(t0 + 1, buf1)
        return carry

    lax.fori_loop(0, NT // 2, ring, 0)


@functools.partial(
    pl.kernel,
    mesh=plsc.VectorSubcoreMesh(core_axis_name="c", subcore_axis_name="s"),
    out_type=jax.ShapeDtypeStruct((B, C), jnp.float32),
    scratch_types=[
        pltpu.VMEM((CCH * HW,), jnp.float32),
        pltpu.VMEM((CCH * HW,), jnp.float32),
        pltpu.VMEM((C,), jnp.float32),
        pltpu.SemaphoreType.DMA,
        pltpu.SemaphoreType.DMA,
    ],
    compiler_params=pltpu.CompilerParams(needs_layout_passes=False),
)
def _sc_pool(x_hbm, out_hbm, buf0, buf1, outbuf, sem0, sem1):
    _sc_pool_body(x_hbm, out_hbm, buf0, buf1, outbuf, sem0, sem1)


def _finish_body(pooled_ref, task_ref, wt_ref, brow_ref,
                 logits_ref, weights_ref, sel_ref, mask_ref):
    t = task_ref[...]
    t = t - jnp.max(t, axis=-1, keepdims=True)
    te = jnp.exp(t)
    tsm = te / jnp.sum(te, axis=-1, keepdims=True)

    fused = jnp.concatenate([pooled_ref[...], tsm], axis=1)
    logits = jnp.dot(fused, wt_ref[...],
                     preferred_element_type=jnp.float32) + brow_ref[...]
    logits_ref[...] = logits

    m = jnp.max(logits, axis=-1, keepdims=True)
    pe = jnp.exp(logits - m)
    probs = pe / jnp.sum(pe, axis=-1, keepdims=True)
    lane = lax.broadcasted_iota(jnp.int32, (B, E), 1)
    v1 = jnp.max(probs, axis=-1, keepdims=True)
    i1 = jnp.min(jnp.where(probs == v1, lane, E), axis=-1, keepdims=True)
    p2 = jnp.where(lane == i1, -jnp.inf, probs)
    v2 = jnp.max(p2, axis=-1, keepdims=True)
    i2 = jnp.min(jnp.where(p2 == v2, lane, E), axis=-1, keepdims=True)
    s12 = v1 + v2
    weights_ref[...] = jnp.concatenate([v1 / s12, v2 / s12], axis=1)
    sel_ref[...] = jnp.concatenate([i1, i2], axis=1)

    selT = jnp.transpose(jnp.concatenate([i1, i2], axis=1), (1, 0))
    e3 = lax.broadcasted_iota(jnp.int32, (E, TOP_K, B), 0)
    mask_ref[...] = (e3 == selT.reshape(1, TOP_K, B)).astype(jnp.int32)


@jax.jit
def _run(hidden_states, task_cls, wt, brow):
    x2 = hidden_states.reshape(B, C * HW)
    pooled = _sc_pool(x2)
    return pl.pallas_call(
        _finish_body,
        out_shape=[
            jax.ShapeDtypeStruct((B, E), jnp.float32),
            jax.ShapeDtypeStruct((B, TOP_K), jnp.float32),
            jax.ShapeDtypeStruct((B, TOP_K), jnp.int32),
            jax.ShapeDtypeStruct((E, TOP_K, B), jnp.int32),
        ],
    )(pooled, task_cls, wt, brow)


def kernel(hidden_states, task_cls, W, b):
    logits, weights, sel, mask = _run(hidden_states, task_cls,
                                      W.T, b.reshape(1, E))
    return (logits, weights, sel, mask)
